# baseline (speedup 1.0000x reference)
import functools

import jax
import jax.numpy as jnp
from jax import lax
from jax.experimental import pallas as pl
from jax.experimental.pallas import tpu as pltpu
from jax.experimental.pallas import tpu_sc as plsc

N_NODES = 10000
N_EDGES = 320000
NPAD = 10016
NC, NS = 2, 16
NW = NC * NS
CHUNK = 128
KIDX = 8
GROUP = KIDX * CHUNK
NG = 20
D0, D1 = 9, 11
TG = NS * NG
EP = TG * GROUP

_mesh = plsc.VectorSubcoreMesh(core_axis_name="c", subcore_axis_name="s")
_sc_params = pltpu.CompilerParams(use_tc_tiling_on_sc=False)


@functools.partial(
    pl.kernel,
    out_type=jax.ShapeDtypeStruct((NC, NPAD), jnp.float32),
    mesh=_mesh,
    compiler_params=_sc_params,
    scratch_types=[
        pltpu.VMEM((2, KIDX, CHUNK), jnp.int32),
        pltpu.VMEM((CHUNK,), jnp.float32),
        pltpu.VMEM_SHARED((NPAD,), jnp.float32),
        pltpu.SemaphoreType.DMA,
    ],
)
def _sc_degree(half_hbm, idx_hbm, out_hbm, idx_v, ones_v, acc_sh, ssem):
    c = lax.axis_index("c")
    s = lax.axis_index("s")

    @pl.when(s == 0)
    def _():
        pltpu.sync_copy(half_hbm, acc_sh)

    for j in range(CHUNK // 16):
        ones_v[pl.ds(j * 16, 16)] = jnp.ones((16,), jnp.float32)
    plsc.subcore_barrier()

    base_g = jnp.where(c == 0, s * D0, NS * D0 + s * D1)
    ng = jnp.where(c == 0, D0, D1)

    def body(g, carry):
        pltpu.sync_copy(idx_hbm.at[base_g + g], idx_v)
        descs = [
            pltpu.async_copy(ones_v, acc_sh.at[idx_v.at[1, j]], ssem,
                             add=True)
            for j in range(KIDX)
        ]
        for d in descs:
            d.wait()
        return carry

    lax.fori_loop(0, ng, body, 0)
    plsc.subcore_barrier()

    @pl.when(s == 0)
    def _():
        pltpu.sync_copy(acc_sh, out_hbm.at[c])


def _make_sc_aggregate(F):
    @functools.partial(
        pl.kernel,
        out_type=jax.ShapeDtypeStruct((N_NODES, F), jnp.float32),
        mesh=_mesh,
        compiler_params=_sc_params,
        scratch_types=[
            pltpu.VMEM((2, KIDX, CHUNK), jnp.int32),
            pltpu.VMEM((KIDX, CHUNK, F), jnp.float32),
            pltpu.VMEM((CHUNK, F), jnp.float32),
            pltpu.VMEM_SHARED((NPAD, F), jnp.float32),
            pltpu.SemaphoreType.DMA((KIDX,)),
            pltpu.SemaphoreType.DMA,
        ],
    )
    def agg(g_hbm, idx_hbm, out_hbm, idx_v, rows_v, zb_v, acc_sh, gsem, ssem):
        c = lax.axis_index("c")
        s = lax.axis_index("s")

        @pl.when(c == 0)
        def _work():
            for i in range(CHUNK):
                for k in range(F // 16):
                    zb_v[i, pl.ds(k * 16, 16)] = jnp.zeros((16,), jnp.float32)
            r0 = s * 624
            for k in range(4):
                pltpu.sync_copy(zb_v, acc_sh.at[pl.ds(r0 + k * CHUNK, CHUNK)])
            pltpu.sync_copy(zb_v.at[pl.ds(0, 112)],
                            acc_sh.at[pl.ds(r0 + 512, 112)])

            @pl.when(s == 0)
            def _():
                pltpu.sync_copy(zb_v.at[pl.ds(0, 32)],
                                acc_sh.at[pl.ds(9984, 32)])

            plsc.subcore_barrier()

            def body(g, carry):
                pltpu.sync_copy(idx_hbm.at[s * NG + g], idx_v)
                gd = [
                    pltpu.async_copy(g_hbm.at[idx_v.at[0, j]], rows_v.at[j],
                                     gsem.at[j])
                    for j in range(KIDX)
                ]
                sd = []
                for j in range(KIDX):
                    gd[j].wait()
                    sd.append(pltpu.async_copy(rows_v.at[j],
                                               acc_sh.at[idx_v.at[1, j]],
                                               ssem, add=True))
                for d in sd:
                    d.wait()
                return carry

            lax.fori_loop(0, NG, body, 0)
            plsc.subcore_barrier()

            pltpu.sync_copy(acc_sh.at[pl.ds(r0, 624)],
                            out_hbm.at[pl.ds(r0, 624)])

            @pl.when(s == 0)
            def _():
                pltpu.sync_copy(acc_sh.at[pl.ds(9984, 16)],
                                out_hbm.at[pl.ds(9984, 16)])

    return agg


_sc_agg64 = _make_sc_aggregate(64)
_sc_agg32 = _make_sc_aggregate(32)


def _tc1_body(dacc_ref, x_ref, w1_ref, g1_ref, dinv_ref):
    deg = dacc_ref[0] + dacc_ref[1]
    dinv = lax.rsqrt(deg)
    dinv_ref[...] = dinv
    h = jnp.dot(x_ref[...], w1_ref[...], preferred_element_type=jnp.float32)
    g1_ref[...] = dinv * h


def _tc2_body(acc_ref, g1_ref, dinv_ref, w2_ref, b1_ref, g2_ref):
    t = acc_ref[...] + g1_ref[...]
    dinv = dinv_ref[...]
    h = jnp.maximum(dinv * t + b1_ref[...], 0.0)
    g2_ref[...] = dinv * jnp.dot(h, w2_ref[...],
                                 preferred_element_type=jnp.float32)


def _tc3_body(acc_ref, g2_ref, dinv_ref, b2_ref, out_ref):
    t = acc_ref[...] + g2_ref[...]
    out_ref[...] = jnp.maximum(dinv_ref[...] * t + b2_ref[...], 0.0)


def kernel(x, edge_index, W1, b1, W2, b2):
    src = edge_index[0].astype(jnp.int32)
    dst = edge_index[1].astype(jnp.int32)
    pad = EP - N_EDGES
    srcp = jnp.concatenate([src, jnp.zeros((pad,), jnp.int32)])
    dstp = jnp.concatenate([dst, jnp.full((pad,), N_NODES, jnp.int32)])
    idx = jnp.stack([srcp.reshape(TG, GROUP),
                     dstp.reshape(TG, GROUP)], axis=1)
    idx = idx.reshape(TG, 2, KIDX, CHUNK)
    half = jnp.full((NPAD,), 0.5, jnp.float32)

    deg_parts = _sc_degree(half, idx)[:, :N_NODES]

    g1, dinv = pl.pallas_call(
        _tc1_body,
        out_shape=[
            jax.ShapeDtypeStruct((N_NODES, 64), jnp.float32),
            jax.ShapeDtypeStruct((N_NODES, 1), jnp.float32),
        ],
    )(deg_parts.reshape(NC, N_NODES, 1), x, W1)

    acc1 = _sc_agg64(g1, idx)

    g2 = pl.pallas_call(
        _tc2_body,
        out_shape=jax.ShapeDtypeStruct((N_NODES, 32), jnp.float32),
    )(acc1, g1, dinv, W2, b1.reshape(1, 64))

    acc2 = _sc_agg32(g2, idx)

    out = pl.pallas_call(
        _tc3_body,
        out_shape=jax.ShapeDtypeStruct((N_NODES, 32), jnp.float32),
    )(acc2, g2, dinv, b2.reshape(1, 32))

    return out


# speedup vs baseline: 1.2622x; 1.2622x over previous
import functools

import jax
import jax.numpy as jnp
from jax import lax
from jax.experimental import pallas as pl
from jax.experimental.pallas import tpu as pltpu
from jax.experimental.pallas import tpu_sc as plsc

N_NODES = 10000
N_EDGES = 320000
NPAD = 10016
NC, NS = 2, 16
NW = NC * NS
CHUNK = 128
KIDX = 8
GROUP = KIDX * CHUNK
G0, G1 = 15, 5
D0, D1 = 9, 11
TG = NS * (G0 + G1)
EP = TG * GROUP

_mesh = plsc.VectorSubcoreMesh(core_axis_name="c", subcore_axis_name="s")
_sc_params = pltpu.CompilerParams(use_tc_tiling_on_sc=False)


@functools.partial(
    pl.kernel,
    out_type=jax.ShapeDtypeStruct((NC, NPAD), jnp.float32),
    mesh=_mesh,
    compiler_params=_sc_params,
    scratch_types=[
        pltpu.VMEM((2, KIDX, CHUNK), jnp.int32),
        pltpu.VMEM((CHUNK,), jnp.float32),
        pltpu.VMEM_SHARED((NPAD,), jnp.float32),
        pltpu.SemaphoreType.DMA,
    ],
)
def _sc_degree(half_hbm, idx_hbm, out_hbm, idx_v, ones_v, acc_sh, ssem):
    c = lax.axis_index("c")
    s = lax.axis_index("s")

    @pl.when(s == 0)
    def _():
        pltpu.sync_copy(half_hbm, acc_sh)

    for j in range(CHUNK // 16):
        ones_v[pl.ds(j * 16, 16)] = jnp.ones((16,), jnp.float32)
    plsc.subcore_barrier()

    base_g = jnp.where(c == 0, s * D0, NS * D0 + s * D1)
    ng = jnp.where(c == 0, D0, D1)

    def body(g, carry):
        pltpu.sync_copy(idx_hbm.at[base_g + g], idx_v)
        descs = [
            pltpu.async_copy(ones_v, acc_sh.at[idx_v.at[1, j]], ssem,
                             add=True)
            for j in range(KIDX)
        ]
        for d in descs:
            d.wait()
        return carry

    lax.fori_loop(0, ng, body, 0)
    plsc.subcore_barrier()

    @pl.when(s == 0)
    def _():
        pltpu.sync_copy(acc_sh, out_hbm.at[c])


def _make_sc_aggregate(F, A0, A1):
    @functools.partial(
        pl.kernel,
        out_type=jax.ShapeDtypeStruct((NC, N_NODES, F), jnp.float32),
        mesh=_mesh,
        compiler_params=_sc_params,
        scratch_types=[
            pltpu.VMEM((2, KIDX, CHUNK), jnp.int32),
            pltpu.VMEM((KIDX, CHUNK, F), jnp.float32),
            pltpu.VMEM((CHUNK, F), jnp.float32),
            pltpu.VMEM_SHARED((NPAD, F), jnp.float32),
            pltpu.SemaphoreType.DMA((KIDX,)),
            pltpu.SemaphoreType.DMA,
        ],
    )
    def agg(g_hbm, idx_hbm, out_hbm, idx_v, rows_v, zb_v, acc_sh, gsem, ssem):
        c = lax.axis_index("c")
        s = lax.axis_index("s")

        for i in range(CHUNK):
            for k in range(F // 16):
                zb_v[i, pl.ds(k * 16, 16)] = jnp.zeros((16,), jnp.float32)
        r0 = s * 624
        for k in range(4):
            pltpu.sync_copy(zb_v, acc_sh.at[pl.ds(r0 + k * CHUNK, CHUNK)])
        pltpu.sync_copy(zb_v.at[pl.ds(0, 112)],
                        acc_sh.at[pl.ds(r0 + 512, 112)])

        @pl.when(s == 0)
        def _():
            pltpu.sync_copy(zb_v.at[pl.ds(0, 32)], acc_sh.at[pl.ds(9984, 32)])

        plsc.subcore_barrier()

        base_g = jnp.where(c == 0, s * A0, NS * A0 + s * A1)
        ng = jnp.where(c == 0, A0, A1)

        def body(g, carry):
            pltpu.sync_copy(idx_hbm.at[base_g + g], idx_v)
            gd = [
                pltpu.async_copy(g_hbm.at[idx_v.at[0, j]], rows_v.at[j],
                                 gsem.at[j])
                for j in range(KIDX)
            ]
            sd = []
            for j in range(KIDX):
                gd[j].wait()
                sd.append(pltpu.async_copy(rows_v.at[j],
                                           acc_sh.at[idx_v.at[1, j]],
                                           ssem, add=True))
            for d in sd:
                d.wait()
            return carry

        lax.fori_loop(0, ng, body, 0)
        plsc.subcore_barrier()

        pltpu.sync_copy(acc_sh.at[pl.ds(r0, 624)],
                        out_hbm.at[c, pl.ds(r0, 624)])

        @pl.when(s == 0)
        def _():
            pltpu.sync_copy(acc_sh.at[pl.ds(9984, 16)],
                            out_hbm.at[c, pl.ds(9984, 16)])

    return agg


_sc_agg64 = _make_sc_aggregate(64, G0, G1)
_sc_agg32 = _make_sc_aggregate(32, G0, G1)


def _tc1_body(dacc_ref, x_ref, w1_ref, g1_ref, dinv_ref):
    deg = dacc_ref[0] + dacc_ref[1]
    dinv = lax.rsqrt(deg)
    dinv_ref[...] = dinv
    h = jnp.dot(x_ref[...], w1_ref[...], preferred_element_type=jnp.float32)
    g1_ref[...] = dinv * h


def _tc2_body(acc_ref, g1_ref, dinv_ref, w2_ref, b1_ref, g2_ref):
    t = acc_ref[0] + acc_ref[1] + g1_ref[...]
    dinv = dinv_ref[...]
    h = jnp.maximum(dinv * t + b1_ref[...], 0.0)
    g2_ref[...] = dinv * jnp.dot(h, w2_ref[...],
                                 preferred_element_type=jnp.float32)


def _tc3_body(acc_ref, g2_ref, dinv_ref, b2_ref, out_ref):
    t = acc_ref[0] + acc_ref[1] + g2_ref[...]
    out_ref[...] = jnp.maximum(dinv_ref[...] * t + b2_ref[...], 0.0)


def kernel(x, edge_index, W1, b1, W2, b2):
    src = edge_index[0].astype(jnp.int32)
    dst = edge_index[1].astype(jnp.int32)
    pad = EP - N_EDGES
    srcp = jnp.concatenate([src, jnp.zeros((pad,), jnp.int32)])
    dstp = jnp.concatenate([dst, jnp.full((pad,), N_NODES, jnp.int32)])
    idx = jnp.stack([srcp.reshape(TG, GROUP),
                     dstp.reshape(TG, GROUP)], axis=1)
    idx = idx.reshape(TG, 2, KIDX, CHUNK)
    half = jnp.full((NPAD,), 0.5, jnp.float32)

    deg_parts = _sc_degree(half, idx)[:, :N_NODES]

    g1, dinv = pl.pallas_call(
        _tc1_body,
        out_shape=[
            jax.ShapeDtypeStruct((N_NODES, 64), jnp.float32),
            jax.ShapeDtypeStruct((N_NODES, 1), jnp.float32),
        ],
    )(deg_parts.reshape(NC, N_NODES, 1), x, W1)

    acc1 = _sc_agg64(g1, idx)

    g2 = pl.pallas_call(
        _tc2_body,
        out_shape=jax.ShapeDtypeStruct((N_NODES, 32), jnp.float32),
    )(acc1, g1, dinv, W2, b1.reshape(1, 64))

    acc2 = _sc_agg32(g2, idx)

    out = pl.pallas_call(
        _tc3_body,
        out_shape=jax.ShapeDtypeStruct((N_NODES, 32), jnp.float32),
    )(acc2, g2, dinv, b2.reshape(1, 32))

    return out


# speedup vs baseline: 1.3002x; 1.0301x over previous
import functools

import jax
import jax.numpy as jnp
from jax import lax
from jax.experimental import pallas as pl
from jax.experimental.pallas import tpu as pltpu
from jax.experimental.pallas import tpu_sc as plsc

N_NODES = 10000
N_EDGES = 320000
NPAD = 10016
NC, NS = 2, 16
NW = NC * NS
CHUNK = 128
KIDX = 8
GROUP = KIDX * CHUNK
G0, G1 = 18, 2
D0, D1 = 11, 9
TG = NS * (G0 + G1)
EP = TG * GROUP

_mesh = plsc.VectorSubcoreMesh(core_axis_name="c", subcore_axis_name="s")
_sc_params = pltpu.CompilerParams(use_tc_tiling_on_sc=False)


@functools.partial(
    pl.kernel,
    out_type=jax.ShapeDtypeStruct((NC, NPAD), jnp.float32),
    mesh=_mesh,
    compiler_params=_sc_params,
    scratch_types=[
        pltpu.VMEM((2, KIDX, CHUNK), jnp.int32),
        pltpu.VMEM((CHUNK,), jnp.float32),
        pltpu.VMEM_SHARED((NPAD,), jnp.float32),
        pltpu.SemaphoreType.DMA,
    ],
)
def _sc_degree(half_hbm, idx_hbm, out_hbm, idx_v, ones_v, acc_sh, ssem):
    c = lax.axis_index("c")
    s = lax.axis_index("s")

    @pl.when(s == 0)
    def _():
        pltpu.sync_copy(half_hbm, acc_sh)

    for j in range(CHUNK // 16):
        ones_v[pl.ds(j * 16, 16)] = jnp.ones((16,), jnp.float32)
    plsc.subcore_barrier()

    base_g = jnp.where(c == 0, s * D0, NS * D0 + s * D1)
    ng = jnp.where(c == 0, D0, D1)

    def body(g, carry):
        pltpu.sync_copy(idx_hbm.at[base_g + g], idx_v)
        descs = [
            pltpu.async_copy(ones_v, acc_sh.at[idx_v.at[1, j]], ssem,
                             add=True)
            for j in range(KIDX)
        ]
        for d in descs:
            d.wait()
        return carry

    lax.fori_loop(0, ng, body, 0)
    plsc.subcore_barrier()

    @pl.when(s == 0)
    def _():
        pltpu.sync_copy(acc_sh, out_hbm.at[c])


def _make_sc_aggregate(F, A0, A1):
    @functools.partial(
        pl.kernel,
        out_type=jax.ShapeDtypeStruct((NC, N_NODES, F), jnp.float32),
        mesh=_mesh,
        compiler_params=_sc_params,
        scratch_types=[
            pltpu.VMEM((2, KIDX, CHUNK), jnp.int32),
            pltpu.VMEM((KIDX, CHUNK, F), jnp.float32),
            pltpu.VMEM((CHUNK, F), jnp.float32),
            pltpu.VMEM_SHARED((NPAD, F), jnp.float32),
            pltpu.SemaphoreType.DMA((KIDX,)),
            pltpu.SemaphoreType.DMA,
        ],
    )
    def agg(g_hbm, idx_hbm, out_hbm, idx_v, rows_v, zb_v, acc_sh, gsem, ssem):
        c = lax.axis_index("c")
        s = lax.axis_index("s")

        for i in range(CHUNK):
            for k in range(F // 16):
                zb_v[i, pl.ds(k * 16, 16)] = jnp.zeros((16,), jnp.float32)
        r0 = s * 624
        for k in range(4):
            pltpu.sync_copy(zb_v, acc_sh.at[pl.ds(r0 + k * CHUNK, CHUNK)])
        pltpu.sync_copy(zb_v.at[pl.ds(0, 112)],
                        acc_sh.at[pl.ds(r0 + 512, 112)])

        @pl.when(s == 0)
        def _():
            pltpu.sync_copy(zb_v.at[pl.ds(0, 32)], acc_sh.at[pl.ds(9984, 32)])

        plsc.subcore_barrier()

        base_g = jnp.where(c == 0, s * A0, NS * A0 + s * A1)
        ng = jnp.where(c == 0, A0, A1)

        def body(g, carry):
            pltpu.sync_copy(idx_hbm.at[base_g + g], idx_v)
            gd = [
                pltpu.async_copy(g_hbm.at[idx_v.at[0, j]], rows_v.at[j],
                                 gsem.at[j])
                for j in range(KIDX)
            ]
            sd = []
            for j in range(KIDX):
                gd[j].wait()
                sd.append(pltpu.async_copy(rows_v.at[j],
                                           acc_sh.at[idx_v.at[1, j]],
                                           ssem, add=True))
            for d in sd:
                d.wait()
            return carry

        lax.fori_loop(0, ng, body, 0)
        plsc.subcore_barrier()

        pltpu.sync_copy(acc_sh.at[pl.ds(r0, 624)],
                        out_hbm.at[c, pl.ds(r0, 624)])

        @pl.when(s == 0)
        def _():
            pltpu.sync_copy(acc_sh.at[pl.ds(9984, 16)],
                            out_hbm.at[c, pl.ds(9984, 16)])

    return agg


_sc_agg64 = _make_sc_aggregate(64, G0, G1)
_sc_agg32 = _make_sc_aggregate(32, G0, G1)


def _tc1_body(dacc_ref, x_ref, w1_ref, g1_ref, dinv_ref):
    deg = dacc_ref[0] + dacc_ref[1]
    dinv = lax.rsqrt(deg)
    dinv_ref[...] = dinv
    h = jnp.dot(x_ref[...], w1_ref[...], preferred_element_type=jnp.float32)
    g1_ref[...] = dinv * h


def _tc2_body(acc_ref, g1_ref, dinv_ref, w2_ref, b1_ref, g2_ref):
    t = acc_ref[0] + acc_ref[1] + g1_ref[...]
    dinv = dinv_ref[...]
    h = jnp.maximum(dinv * t + b1_ref[...], 0.0)
    g2_ref[...] = dinv * jnp.dot(h, w2_ref[...],
                                 preferred_element_type=jnp.float32)


def _tc3_body(acc_ref, g2_ref, dinv_ref, b2_ref, out_ref):
    t = acc_ref[0] + acc_ref[1] + g2_ref[...]
    out_ref[...] = jnp.maximum(dinv_ref[...] * t + b2_ref[...], 0.0)


def kernel(x, edge_index, W1, b1, W2, b2):
    src = edge_index[0].astype(jnp.int32)
    dst = edge_index[1].astype(jnp.int32)
    pad = EP - N_EDGES
    srcp = jnp.concatenate([src, jnp.zeros((pad,), jnp.int32)])
    dstp = jnp.concatenate([dst, jnp.full((pad,), N_NODES, jnp.int32)])
    idx = jnp.stack([srcp.reshape(TG, GROUP),
                     dstp.reshape(TG, GROUP)], axis=1)
    idx = idx.reshape(TG, 2, KIDX, CHUNK)
    half = jnp.full((NPAD,), 0.5, jnp.float32)

    deg_parts = _sc_degree(half, idx)[:, :N_NODES]

    g1, dinv = pl.pallas_call(
        _tc1_body,
        out_shape=[
            jax.ShapeDtypeStruct((N_NODES, 64), jnp.float32),
            jax.ShapeDtypeStruct((N_NODES, 1), jnp.float32),
        ],
    )(deg_parts.reshape(NC, N_NODES, 1), x, W1)

    acc1 = _sc_agg64(g1, idx)

    g2 = pl.pallas_call(
        _tc2_body,
        out_shape=jax.ShapeDtypeStruct((N_NODES, 32), jnp.float32),
    )(acc1, g1, dinv, W2, b1.reshape(1, 64))

    acc2 = _sc_agg32(g2, idx)

    out = pl.pallas_call(
        _tc3_body,
        out_shape=jax.ShapeDtypeStruct((N_NODES, 32), jnp.float32),
    )(acc2, g2, dinv, b2.reshape(1, 32))

    return out
